# baseline (speedup 1.0000x reference)
import functools

import jax
import jax.numpy as jnp
from jax import lax
from jax.experimental import pallas as pl
from jax.experimental.pallas import tpu as pltpu
from jax.experimental.pallas import tpu_sc as plsc

B = 16384
F = 26
D = 32
LANES = 16
SUPER = 128

CHUNK = 16
NIDX = CHUNK * F
IDX_W = 104
NSTREAM = NIDX // IDX_W


def _make_kernel(num_workers):
    rows_per_w = B // num_workers
    nchunks = rows_per_w // CHUNK

    mesh = plsc.VectorSubcoreMesh(core_axis_name="c", subcore_axis_name="s")

    @functools.partial(
        pl.kernel,
        mesh=mesh,
        out_type=jax.ShapeDtypeStruct((D, B), jnp.float32),
        compiler_params=pltpu.CompilerParams(
            use_tc_tiling_on_sc=False, needs_layout_passes=False),
        scratch_types=[
            pltpu.VMEM((F, rows_per_w), jnp.int32),
            pltpu.VMEM((NIDX,), jnp.int32),
            pltpu.VMEM((NIDX,), jnp.int32),
            pltpu.VMEM((NIDX, SUPER), jnp.float32),
            pltpu.VMEM((D, CHUNK), jnp.float32),
            pltpu.SemaphoreType.DMA,
        ],
    )
    def emb_kernel(feats_hbm, table_hbm, out_hbm, idx_v, sup0,
                   col0, rows0, out_v, sem0):
        num_cores = lax.axis_size("c")
        wid = lax.axis_index("s") * num_cores + lax.axis_index("c")
        b0 = wid * rows_per_w

        pltpu.sync_copy(feats_hbm.at[:, pl.ds(b0, rows_per_w)], idx_v)

        sups = (sup0,)
        cols = (col0,)
        bufs = (rows0,)
        sems = (sem0,)

        def fire(k, par):
            sup = sups[par]
            col = cols[par]
            sem = sems[par]
            for f in range(F):
                raw = idx_v[f, pl.ds(k * CHUNK, CHUNK)]
                sup[pl.ds(f * CHUNK, CHUNK)] = lax.shift_right_logical(raw, 2)
                col[pl.ds(f * CHUNK, CHUNK)] = (raw & 3) * D
            for j in range(NSTREAM):
                pltpu.async_copy(
                    table_hbm.at[sup.at[pl.ds(j * IDX_W, IDX_W)]],
                    bufs[par].at[pl.ds(j * IDX_W, IDX_W)],
                    sem,
                )

        def drain(par):
            for j in range(NSTREAM):
                pltpu.make_async_copy(
                    table_hbm.at[pl.ds(0, IDX_W)],
                    bufs[par].at[pl.ds(j * IDX_W, IDX_W)],
                    sems[par],
                ).wait()

        def reduce(k, par):
            buf = bufs[par]
            col = cols[par]
            iota = lax.iota(jnp.int32, LANES)
            dg = 8
            for d0 in range(0, D, dg):
                accs = [None] * dg
                for f in range(F):
                    rows = iota + (f * CHUNK)
                    cols0 = col[pl.ds(f * CHUNK, CHUNK)]
                    for i in range(dg):
                        v = plsc.load_gather(buf, [rows, cols0 + (d0 + i)])
                        accs[i] = v if f == 0 else accs[i] + v
                for i in range(dg):
                    out_v[d0 + i, :] = accs[i]
            pltpu.sync_copy(out_v, out_hbm.at[:, pl.ds(b0 + k * CHUNK, CHUNK)])

        def body(k, _):
            fire(k, 0)
            drain(0)
            reduce(k, 0)
            return _

        lax.fori_loop(0, nchunks, body, None)

    return emb_kernel


def kernel(categorical_feats, table):
    info = plsc.get_sparse_core_info()
    num_workers = info.num_cores * info.num_subcores
    feats_t = categorical_feats.T.astype(jnp.int32)
    table_r = table.reshape(table.shape[0] * D // SUPER, SUPER)
    out_t = _make_kernel(num_workers)(feats_t, table_r)
    return out_t.T
